# Initial kernel scaffold; baseline (speedup 1.0000x reference)
import jax
import jax.numpy as jnp
from jax.experimental import pallas as pl


def kernel(e, q, H, r_space, e_space, action_mask, entity_emb, relation_emb, W1, b1, W2, b2):
    raise NotImplementedError("write your pallas kernel here")


# speedup vs baseline: 2.7125x; 2.7125x over previous
import functools

import jax
import jax.numpy as jnp
from jax import lax
from jax.experimental import pallas as pl
from jax.experimental.pallas import tpu as pltpu
from jax.experimental.pallas import tpu_sc as plsc

HUGE_INT = 1e31
EPS = 1e-20

B, A = 1024, 200
ED, RD, HD = 128, 128, 256
NR = 400
A_PAD = 208
NW = 32
BPW = B // NW
NBLK = A_PAD // 16

_INTERPRET = False


def _wid():
    return lax.axis_index("s") * 2 + lax.axis_index("c")


def _sc_gather_eq(e, q, entity_emb, relation_emb):
    mesh = plsc.VectorSubcoreMesh(core_axis_name="c", subcore_axis_name="s", num_cores=2, num_subcores=16)

    @functools.partial(
        pl.kernel,
        out_type=(jax.ShapeDtypeStruct((B, ED), jnp.float32),
                  jax.ShapeDtypeStruct((B, RD), jnp.float32)),
        mesh=mesh,
        interpret=_INTERPRET,
        scratch_types=[
            pltpu.VMEM((BPW,), jnp.int32),
            pltpu.VMEM((BPW, ED), jnp.float32),
            pltpu.VMEM((BPW,), jnp.int32),
            pltpu.VMEM((BPW, RD), jnp.float32),
            pltpu.SemaphoreType.DMA,
            pltpu.SemaphoreType.DMA,
        ],
    )
    def k(e_hbm, q_hbm, ent_hbm, rel_hbm, eout_hbm, qout_hbm,
          eidx_v, erow_v, qidx_v, qrow_v, esem, qsem):
        base = _wid() * BPW
        pltpu.sync_copy(e_hbm.at[pl.ds(base, BPW)], eidx_v)
        pltpu.sync_copy(q_hbm.at[pl.ds(base, BPW)], qidx_v)
        ce = pltpu.async_copy(ent_hbm.at[eidx_v], erow_v, esem)
        cq = pltpu.async_copy(rel_hbm.at[qidx_v], qrow_v, qsem)
        ce.wait()
        cq.wait()
        pltpu.sync_copy(erow_v, eout_hbm.at[pl.ds(base, BPW)])
        pltpu.sync_copy(qrow_v, qout_hbm.at[pl.ds(base, BPW)])

    return k(e, q, entity_emb, relation_emb)


def _tc_mlp(E, H, Q, W1a, W1b, W1c, b1, W2e, W2r, b2e, b2r, relT):
    BB = 256
    prec = lax.Precision.HIGHEST

    def body(e_ref, h_ref, q_ref, w1a_ref, w1b_ref, w1c_ref, b1_ref,
             w2e_ref, w2r_ref, b2e_ref, b2r_ref, relt_ref,
             x2e_ref, reltab_ref):
        x = (jnp.dot(e_ref[...], w1a_ref[...], precision=prec)
             + jnp.dot(h_ref[...], w1b_ref[...], precision=prec)
             + jnp.dot(q_ref[...], w1c_ref[...], precision=prec)
             + b1_ref[...])
        x = jnp.maximum(x, 0.0)
        x2e_ref[...] = jnp.dot(x, w2e_ref[...], precision=prec) + b2e_ref[...]
        x2r = jnp.dot(x, w2r_ref[...], precision=prec) + b2r_ref[...]
        reltab_ref[...] = jnp.dot(x2r, relt_ref[...], precision=prec)

    full = lambda s: pl.BlockSpec(s, lambda i: (0, 0))
    return pl.pallas_call(
        body,
        grid=(B // BB,),
        in_specs=[
            pl.BlockSpec((BB, ED), lambda i: (i, 0)),
            pl.BlockSpec((BB, HD), lambda i: (i, 0)),
            pl.BlockSpec((BB, RD), lambda i: (i, 0)),
            full((ED, 256)), full((HD, 256)), full((RD, 256)), full((1, 256)),
            full((256, ED)), full((256, RD)), full((1, ED)), full((1, RD)),
            full((RD, NR)),
        ],
        out_specs=[
            pl.BlockSpec((BB, ED), lambda i: (i, 0)),
            pl.BlockSpec((BB, NR), lambda i: (i, 0)),
        ],
        out_shape=[
            jax.ShapeDtypeStruct((B, ED), jnp.float32),
            jax.ShapeDtypeStruct((B, NR), jnp.float32),
        ],
        interpret=_INTERPRET,
    )(E, H, Q, W1a, W1b, W1c, b1, W2e, W2r, b2e, b2r, relT)


def _sc_logits(e_space1d, r_space1d, x2e1d, reltab1d, entity_emb):
    mesh = plsc.VectorSubcoreMesh(core_axis_name="c", subcore_axis_name="s", num_cores=2, num_subcores=16)

    @functools.partial(
        pl.kernel,
        out_type=jax.ShapeDtypeStruct((B * A_PAD,), jnp.float32),
        mesh=mesh,
        interpret=_INTERPRET,
        compiler_params=pltpu.CompilerParams(needs_layout_passes=False),
        scratch_types=[
            pltpu.VMEM((BPW * A,), jnp.int32),
            pltpu.VMEM((BPW * A + 64,), jnp.int32),
            pltpu.VMEM((BPW * ED,), jnp.float32),
            pltpu.VMEM((BPW * NR,), jnp.float32),
            pltpu.VMEM((A_PAD, ED), jnp.float32),
            pltpu.VMEM((A_PAD, ED), jnp.float32),
            pltpu.VMEM((BPW * A_PAD,), jnp.float32),
            pltpu.SemaphoreType.DMA,
            pltpu.SemaphoreType.DMA,
        ],
    )
    def k(es_hbm, rs_hbm, x2e_hbm, relt_hbm, ent_hbm, out_hbm,
          es_v, rs_v, x2e_v, relt_v, rows0, rows1, out_v, sem0, sem1):
        base = _wid() * BPW
        pltpu.sync_copy(es_hbm.at[pl.ds(base * A, BPW * A)],
                        es_v)
        pltpu.sync_copy(rs_hbm.at[pl.ds(base * A, BPW * A)],
                        rs_v.at[pl.ds(0, BPW * A)])
        pltpu.sync_copy(x2e_hbm.at[pl.ds(base * ED, BPW * ED)], x2e_v)
        pltpu.sync_copy(relt_hbm.at[pl.ds(base * NR, BPW * NR)], relt_v)

        iota = lax.iota(jnp.int32, 16)
        aidx = [iota + (kk * 16) for kk in range(NBLK)]

        def issue(i, buf, sem):
            c1 = pltpu.async_copy(ent_hbm.at[es_v.at[pl.ds(i * A, 104)]],
                                  buf.at[pl.ds(0, 104)], sem)
            c2 = pltpu.async_copy(ent_hbm.at[es_v.at[pl.ds(i * A + 104, 96)]],
                                  buf.at[pl.ds(104, 96)], sem)
            del c1, c2

        def wait(buf, sem):
            pltpu.make_async_copy(ent_hbm.at[pl.ds(0, 104)],
                                  buf.at[pl.ds(0, 104)], sem).wait()
            pltpu.make_async_copy(ent_hbm.at[pl.ds(0, 96)],
                                  buf.at[pl.ds(104, 96)], sem).wait()

        def compute(i, buf):
            rel_base = i * NR
            x2_base = i * ED
            accs = []
            for kk in range(NBLK):
                rsv = rs_v[pl.ds(i * A + kk * 16, 16)]
                rsv = jnp.minimum(jnp.maximum(rsv, 0), NR - 1)
                accs.append(plsc.load_gather(relt_v, [rel_base + rsv]))

            def d_body(dd, accs):
                dsp = jnp.full((16,), dd, dtype=jnp.int32)
                xb = plsc.load_gather(
                    x2e_v, [jnp.full((16,), x2_base + dd, dtype=jnp.int32)])
                out = []
                for kk in range(NBLK):
                    col = plsc.load_gather(buf, [aidx[kk], dsp])
                    out.append(accs[kk] + col * xb)
                return tuple(out)

            accs = lax.fori_loop(0, ED, d_body, tuple(accs))
            for kk in range(NBLK):
                out_v[pl.ds(i * A_PAD + kk * 16, 16)] = accs[kk]

        issue(0, rows0, sem0)
        issue(1, rows1, sem1)

        def pair_body(t, carry):
            g0 = 2 * t
            wait(rows0, sem0)
            compute(g0, rows0)
            issue(lax.rem(g0 + 2, BPW), rows0, sem0)
            wait(rows1, sem1)
            compute(g0 + 1, rows1)
            issue(lax.rem(g0 + 3, BPW), rows1, sem1)
            return carry

        lax.fori_loop(0, BPW // 2, pair_body, 0)
        wait(rows0, sem0)
        wait(rows1, sem1)

        pltpu.sync_copy(out_v, out_hbm.at[pl.ds(base * A_PAD, BPW * A_PAD)])

    return k(e_space1d, r_space1d, x2e1d, reltab1d, entity_emb)


def _tc_softmax(logits208, action_mask):
    BB = 256

    def body(lg_ref, mask_ref, p_ref, ent_ref):
        lg = lg_ref[...][:, :A]
        mask = mask_ref[...]
        lgm = lg - (1.0 - mask) * HUGE_INT
        m = jnp.max(lgm, axis=1, keepdims=True)
        ez = jnp.exp(lgm - m)
        s = jnp.sum(ez, axis=1, keepdims=True)
        p = ez / s
        p_ref[...] = p
        ent_ref[...] = -jnp.sum(p * jnp.log(p + EPS), axis=1, keepdims=True)

    return pl.pallas_call(
        body,
        grid=(B // BB,),
        in_specs=[
            pl.BlockSpec((BB, A_PAD), lambda i: (i, 0)),
            pl.BlockSpec((BB, A), lambda i: (i, 0)),
        ],
        out_specs=[
            pl.BlockSpec((BB, A), lambda i: (i, 0)),
            pl.BlockSpec((BB, 1), lambda i: (i, 0)),
        ],
        out_shape=[
            jax.ShapeDtypeStruct((B, A), jnp.float32),
            jax.ShapeDtypeStruct((B, 1), jnp.float32),
        ],
        interpret=_INTERPRET,
    )(logits208, action_mask)


def kernel(e, q, H, r_space, e_space, action_mask, entity_emb, relation_emb,
           W1, b1, W2, b2):
    E, Q = _sc_gather_eq(e, q, entity_emb, relation_emb)

    W1a, W1b, W1c = W1[:ED], W1[ED:ED + HD], W1[ED + HD:]
    x2e, reltab = _tc_mlp(
        E, H, Q, W1a, W1b, W1c, b1.reshape(1, -1),
        W2[:, RD:], W2[:, :RD], b2[RD:].reshape(1, -1), b2[:RD].reshape(1, -1),
        relation_emb.T)

    logits1d = _sc_logits(
        e_space.reshape(-1), r_space.reshape(-1), x2e.reshape(-1),
        reltab.reshape(-1), entity_emb)

    p, ent = _tc_softmax(logits1d.reshape(B, A_PAD), action_mask)
    return p, ent.reshape(B)


# speedup vs baseline: 13.0229x; 4.8010x over previous
import functools

import jax
import jax.numpy as jnp
from jax import lax
from jax.experimental import pallas as pl
from jax.experimental.pallas import tpu as pltpu
from jax.experimental.pallas import tpu_sc as plsc

HUGE_INT = 1e31
EPS = 1e-20

B, A = 1024, 200
ED, RD, HD = 128, 128, 256
NR = 400
A_PAD = 208
NW = 32
BPW = B // NW
NBLK = A_PAD // 16

_INTERPRET = False


def _wid():
    return lax.axis_index("s") * 2 + lax.axis_index("c")


def _sc_gather_eq(e, q, entity_emb, relation_emb):
    mesh = plsc.VectorSubcoreMesh(core_axis_name="c", subcore_axis_name="s", num_cores=2, num_subcores=16)

    @functools.partial(
        pl.kernel,
        out_type=(jax.ShapeDtypeStruct((B, ED), jnp.float32),
                  jax.ShapeDtypeStruct((B, RD), jnp.float32)),
        mesh=mesh,
        interpret=_INTERPRET,
        scratch_types=[
            pltpu.VMEM((BPW,), jnp.int32),
            pltpu.VMEM((BPW, ED), jnp.float32),
            pltpu.VMEM((BPW,), jnp.int32),
            pltpu.VMEM((BPW, RD), jnp.float32),
            pltpu.SemaphoreType.DMA,
            pltpu.SemaphoreType.DMA,
        ],
    )
    def k(e_hbm, q_hbm, ent_hbm, rel_hbm, eout_hbm, qout_hbm,
          eidx_v, erow_v, qidx_v, qrow_v, esem, qsem):
        base = _wid() * BPW
        pltpu.sync_copy(e_hbm.at[pl.ds(base, BPW)], eidx_v)
        pltpu.sync_copy(q_hbm.at[pl.ds(base, BPW)], qidx_v)
        ce = pltpu.async_copy(ent_hbm.at[eidx_v], erow_v, esem)
        cq = pltpu.async_copy(rel_hbm.at[qidx_v], qrow_v, qsem)
        ce.wait()
        cq.wait()
        pltpu.sync_copy(erow_v, eout_hbm.at[pl.ds(base, BPW)])
        pltpu.sync_copy(qrow_v, qout_hbm.at[pl.ds(base, BPW)])

    return k(e, q, entity_emb, relation_emb)


def _tc_mlp(E, H, Q, W1a, W1b, W1c, b1, W2e, W2r, b2e, b2r, relT):
    BB = 256
    prec = lax.Precision.HIGHEST

    def body(e_ref, h_ref, q_ref, w1a_ref, w1b_ref, w1c_ref, b1_ref,
             w2e_ref, w2r_ref, b2e_ref, b2r_ref, relt_ref,
             x2e_ref, reltab_ref):
        x = (jnp.dot(e_ref[...], w1a_ref[...], precision=prec)
             + jnp.dot(h_ref[...], w1b_ref[...], precision=prec)
             + jnp.dot(q_ref[...], w1c_ref[...], precision=prec)
             + b1_ref[...])
        x = jnp.maximum(x, 0.0)
        x2e_ref[...] = jnp.dot(x, w2e_ref[...], precision=prec) + b2e_ref[...]
        x2r = jnp.dot(x, w2r_ref[...], precision=prec) + b2r_ref[...]
        reltab_ref[...] = jnp.dot(x2r, relt_ref[...], precision=prec)

    full = lambda s: pl.BlockSpec(s, lambda i: (0, 0))
    return pl.pallas_call(
        body,
        grid=(B // BB,),
        in_specs=[
            pl.BlockSpec((BB, ED), lambda i: (i, 0)),
            pl.BlockSpec((BB, HD), lambda i: (i, 0)),
            pl.BlockSpec((BB, RD), lambda i: (i, 0)),
            full((ED, 256)), full((HD, 256)), full((RD, 256)), full((1, 256)),
            full((256, ED)), full((256, RD)), full((1, ED)), full((1, RD)),
            full((RD, NR)),
        ],
        out_specs=[
            pl.BlockSpec((BB, ED), lambda i: (i, 0)),
            pl.BlockSpec((BB, NR), lambda i: (i, 0)),
        ],
        out_shape=[
            jax.ShapeDtypeStruct((B, ED), jnp.float32),
            jax.ShapeDtypeStruct((B, NR), jnp.float32),
        ],
        interpret=_INTERPRET,
    )(E, H, Q, W1a, W1b, W1c, b1, W2e, W2r, b2e, b2r, relT)


def _sc_logits(e_space1d, r_space1d, x2e1d, reltab1d, entity_emb):
    mesh = plsc.VectorSubcoreMesh(core_axis_name="c", subcore_axis_name="s", num_cores=2, num_subcores=16)

    @functools.partial(
        pl.kernel,
        out_type=jax.ShapeDtypeStruct((B * A_PAD,), jnp.float32),
        mesh=mesh,
        interpret=_INTERPRET,
        compiler_params=pltpu.CompilerParams(needs_layout_passes=False),
        scratch_types=[
            pltpu.VMEM((BPW * A,), jnp.int32),
            pltpu.VMEM((BPW * A + 64,), jnp.int32),
            pltpu.VMEM((BPW * ED,), jnp.float32),
            pltpu.VMEM((BPW * NR,), jnp.float32),
            pltpu.VMEM((A_PAD, ED), jnp.float32),
            pltpu.VMEM((A_PAD, ED), jnp.float32),
            pltpu.VMEM((BPW * A_PAD,), jnp.float32),
            pltpu.SemaphoreType.DMA,
            pltpu.SemaphoreType.DMA,
        ],
    )
    def k(es_hbm, rs_hbm, x2e_hbm, relt_hbm, ent_hbm, out_hbm,
          es_v, rs_v, x2e_v, relt_v, rows0, rows1, out_v, sem0, sem1):
        base = _wid() * BPW
        pltpu.sync_copy(es_hbm.at[pl.ds(base * A, BPW * A)],
                        es_v)
        pltpu.sync_copy(rs_hbm.at[pl.ds(base * A, BPW * A)],
                        rs_v.at[pl.ds(0, BPW * A)])
        pltpu.sync_copy(x2e_hbm.at[pl.ds(base * ED, BPW * ED)], x2e_v)
        pltpu.sync_copy(relt_hbm.at[pl.ds(base * NR, BPW * NR)], relt_v)

        iota = lax.iota(jnp.int32, 16)
        mask15 = iota == 15

        def issue(i, buf, sem):
            c1 = pltpu.async_copy(ent_hbm.at[es_v.at[pl.ds(i * A, 104)]],
                                  buf.at[pl.ds(0, 104)], sem)
            c2 = pltpu.async_copy(ent_hbm.at[es_v.at[pl.ds(i * A + 104, 96)]],
                                  buf.at[pl.ds(104, 96)], sem)
            del c1, c2

        def wait(buf, sem):
            pltpu.make_async_copy(ent_hbm.at[pl.ds(0, 104)],
                                  buf.at[pl.ds(0, 104)], sem).wait()
            pltpu.make_async_copy(ent_hbm.at[pl.ds(0, 96)],
                                  buf.at[pl.ds(104, 96)], sem).wait()

        def compute(i, buf):
            xv = [x2e_v[pl.ds(i * ED + c * 16, 16)] for c in range(ED // 16)]

            def blk_body(blk, carry):
                a0 = blk * 16
                accs = [buf[a0 + j, pl.ds(0, 16)] * xv[0] for j in range(16)]
                for c in range(1, ED // 16):
                    for j in range(16):
                        accs[j] = accs[j] + buf[a0 + j, pl.ds(c * 16, 16)] * xv[c]
                for j in range(16):
                    cs = plsc.cumsum(accs[j])
                    plsc.store_scatter(
                        out_v,
                        [jnp.full((16,), i * A_PAD + a0 + j, dtype=jnp.int32)],
                        cs, mask=mask15)
                rsv = rs_v[pl.ds(i * A + a0, 16)]
                rsv = jnp.minimum(jnp.maximum(rsv, 0), NR - 1)
                rel = plsc.load_gather(relt_v, [i * NR + rsv])
                pos = i * A_PAD + a0
                out_v[pl.ds(pos, 16)] = out_v[pl.ds(pos, 16)] + rel
                return carry

            lax.fori_loop(0, NBLK, blk_body, 0)

        issue(0, rows0, sem0)
        issue(1, rows1, sem1)

        def pair_body(t, carry):
            g0 = 2 * t
            wait(rows0, sem0)
            compute(g0, rows0)
            issue(lax.rem(g0 + 2, BPW), rows0, sem0)
            wait(rows1, sem1)
            compute(g0 + 1, rows1)
            issue(lax.rem(g0 + 3, BPW), rows1, sem1)
            return carry

        lax.fori_loop(0, BPW // 2, pair_body, 0)
        wait(rows0, sem0)
        wait(rows1, sem1)

        pltpu.sync_copy(out_v, out_hbm.at[pl.ds(base * A_PAD, BPW * A_PAD)])

    return k(e_space1d, r_space1d, x2e1d, reltab1d, entity_emb)


def _tc_softmax(logits208, action_mask):
    BB = 256

    def body(lg_ref, mask_ref, p_ref, ent_ref):
        lg = lg_ref[...][:, :A]
        mask = mask_ref[...]
        lgm = lg - (1.0 - mask) * HUGE_INT
        m = jnp.max(lgm, axis=1, keepdims=True)
        ez = jnp.exp(lgm - m)
        s = jnp.sum(ez, axis=1, keepdims=True)
        p = ez / s
        p_ref[...] = p
        ent_ref[...] = -jnp.sum(p * jnp.log(p + EPS), axis=1, keepdims=True)

    return pl.pallas_call(
        body,
        grid=(B // BB,),
        in_specs=[
            pl.BlockSpec((BB, A_PAD), lambda i: (i, 0)),
            pl.BlockSpec((BB, A), lambda i: (i, 0)),
        ],
        out_specs=[
            pl.BlockSpec((BB, A), lambda i: (i, 0)),
            pl.BlockSpec((BB, 1), lambda i: (i, 0)),
        ],
        out_shape=[
            jax.ShapeDtypeStruct((B, A), jnp.float32),
            jax.ShapeDtypeStruct((B, 1), jnp.float32),
        ],
        interpret=_INTERPRET,
    )(logits208, action_mask)


def kernel(e, q, H, r_space, e_space, action_mask, entity_emb, relation_emb,
           W1, b1, W2, b2):
    E, Q = _sc_gather_eq(e, q, entity_emb, relation_emb)

    W1a, W1b, W1c = W1[:ED], W1[ED:ED + HD], W1[ED + HD:]
    x2e, reltab = _tc_mlp(
        E, H, Q, W1a, W1b, W1c, b1.reshape(1, -1),
        W2[:, RD:], W2[:, :RD], b2[RD:].reshape(1, -1), b2[:RD].reshape(1, -1),
        relation_emb.T)

    logits1d = _sc_logits(
        e_space.reshape(-1), r_space.reshape(-1), x2e.reshape(-1),
        reltab.reshape(-1), entity_emb)

    p, ent = _tc_softmax(logits1d.reshape(B, A_PAD), action_mask)
    return p, ent.reshape(B)
